# Initial kernel scaffold; baseline (speedup 1.0000x reference)
import jax
import jax.numpy as jnp
from jax.experimental import pallas as pl


def kernel(x, pos, edge_index, edge_attr, batch, W_embed, b_embed, We1, be1, We2, be2, Wn, bn, Wie1, bie1, Wie2, bie2, Winv, binv, g1, beta1, W1, b1, g2, beta2, W2, b2):
    raise NotImplementedError("write your pallas kernel here")


# speedup vs baseline: 1.5431x; 1.5431x over previous
import functools

import jax
import jax.numpy as jnp
from jax import lax
from jax.experimental import pallas as pl
from jax.experimental.pallas import tpu as pltpu
from jax.experimental.pallas import tpu_sc as plsc

N = 50000
E = 800000
F = 48
INV = 64
NL = 3
G = 4096
EIN = 8
HID = 64

NP = 51200
CHUNK = 400
NSUB = 16
NCORE = 2

def _f32(shape):
    return jax.ShapeDtypeStruct(shape, jnp.float32)


def _sc_gather_pos_body(pos8, src, dst, out_s, out_d, idx_v, rows_v, sem):
    cid = lax.axis_index("c")
    sid = lax.axis_index("s")
    nchunks = E // CHUNK

    def body(j):
        chunk = sid + NSUB * j

        @pl.when(chunk < nchunks)
        def _():
            base = chunk * CHUNK

            @pl.when(cid == 0)
            def _():
                pltpu.sync_copy(src.at[pl.ds(base, CHUNK)], idx_v)
                pltpu.async_copy(pos8.at[idx_v], rows_v, sem).wait()
                pltpu.sync_copy(rows_v, out_s.at[pl.ds(base, CHUNK)])

            @pl.when(cid == 1)
            def _():
                pltpu.sync_copy(dst.at[pl.ds(base, CHUNK)], idx_v)
                pltpu.async_copy(pos8.at[idx_v], rows_v, sem).wait()
                pltpu.sync_copy(rows_v, out_d.at[pl.ds(base, CHUNK)])

    pl.loop(0, nchunks // NSUB)(body)


def _sc_gather_h_body(h, src, out, idx_v, rows_v, sem):
    cid = lax.axis_index("c")
    sid = lax.axis_index("s")
    nchunks = E // CHUNK
    per_core = nchunks // NCORE

    def body(j):
        local = sid + NSUB * j

        @pl.when(local < per_core)
        def _():
            base = (cid * per_core + local) * CHUNK
            pltpu.sync_copy(src.at[pl.ds(base, CHUNK)], idx_v)
            pltpu.async_copy(h.at[idx_v], rows_v, sem).wait()
            pltpu.sync_copy(rows_v, out.at[pl.ds(base, CHUNK)])

    pl.loop(0, (per_core + NSUB - 1) // NSUB)(body)


def _sc_scatter_body(msga, msgb, dst, zfeed, outa, outb, idx_v, msg_v, z_v, acc, sem):
    cid = lax.axis_index("c")
    sid = lax.axis_index("s")

    pltpu.sync_copy(zfeed, z_v)

    def zbody(j):
        chunk = sid + NSUB * j
        pltpu.sync_copy(z_v, acc.at[pl.ds(chunk * CHUNK, CHUNK)])

    pl.loop(0, NP // CHUNK // NSUB)(zbody)
    plsc.subcore_barrier()

    nchunks = E // CHUNK

    def body(j):
        chunk = sid + NSUB * j

        @pl.when(chunk < nchunks)
        def _():
            base = chunk * CHUNK
            pltpu.sync_copy(dst.at[pl.ds(base, CHUNK)], idx_v)

            @pl.when(cid == 0)
            def _():
                pltpu.sync_copy(msga.at[pl.ds(base, CHUNK)], msg_v)

            @pl.when(cid == 1)
            def _():
                pltpu.sync_copy(msgb.at[pl.ds(base, CHUNK)], msg_v)

            pltpu.sync_copy(msg_v, acc.at[idx_v], add=True)

    pl.loop(0, nchunks // NSUB)(body)
    plsc.subcore_barrier()

    rows = NP // NSUB
    base = sid * rows

    @pl.when(cid == 0)
    def _():
        pltpu.sync_copy(acc.at[pl.ds(base, rows)], outa.at[pl.ds(base, rows)])

    @pl.when(cid == 1)
    def _():
        pltpu.sync_copy(acc.at[pl.ds(base, rows)], outb.at[pl.ds(base, rows)])


def _sc_hist_body(dst, batch, ofeed, zfeed8, deg_out, cnt_out, idx_v, ones_v, acc, sem):
    cid = lax.axis_index("c")
    sid = lax.axis_index("s")

    pltpu.sync_copy(zfeed8, ones_v)

    def zbody(j):
        chunk = sid + NSUB * j

        @pl.when(cid == 0)
        def _():
            pltpu.sync_copy(ones_v, acc.at[pl.ds(chunk * CHUNK, CHUNK)])

        @pl.when(jnp.logical_and(cid == 1, chunk < G // CHUNK + 1))
        def _():
            @pl.when(chunk * CHUNK < G - CHUNK + 1)
            def _():
                pltpu.sync_copy(ones_v, acc.at[pl.ds(chunk * CHUNK, CHUNK)])

            @pl.when(chunk == G // CHUNK)
            def _():
                pltpu.sync_copy(ones_v.at[pl.ds(0, G % CHUNK)],
                                acc.at[pl.ds(G - (G % CHUNK), G % CHUNK)])

    pl.loop(0, NP // CHUNK // NSUB)(zbody)
    pltpu.sync_copy(ofeed, ones_v)
    plsc.subcore_barrier()

    ne = E // CHUNK
    nn = N // CHUNK

    def body(j):
        chunk = sid + NSUB * j

        @pl.when(cid == 0)
        def _():
            pltpu.sync_copy(dst.at[pl.ds(chunk * CHUNK, CHUNK)], idx_v)
            pltpu.sync_copy(ones_v, acc.at[idx_v], add=True)

        @pl.when(jnp.logical_and(cid == 1, chunk < nn))
        def _():
            pltpu.sync_copy(batch.at[pl.ds(chunk * CHUNK, CHUNK)], idx_v)
            pltpu.sync_copy(ones_v, acc.at[idx_v], add=True)

    pl.loop(0, ne // NSUB)(body)
    plsc.subcore_barrier()

    @pl.when(cid == 0)
    def _():
        rows = NP // NSUB
        pltpu.sync_copy(acc.at[pl.ds(sid * rows, rows)],
                        deg_out.at[pl.ds(sid * rows, rows)])

    @pl.when(cid == 1)
    def _():
        rows = G // NSUB
        pltpu.sync_copy(acc.at[pl.ds(sid * rows, rows)],
                        cnt_out.at[pl.ds(sid * rows, rows)])


def _sc_pool_body(inva, invb, batch, zfeed, outa, outb, idx_v, row_v, z_v, acc, sem):
    cid = lax.axis_index("c")
    sid = lax.axis_index("s")

    pltpu.sync_copy(zfeed, z_v)
    rows = G // NSUB
    pltpu.sync_copy(z_v.at[pl.ds(0, rows)], acc.at[pl.ds(sid * rows, rows)])
    plsc.subcore_barrier()

    nchunks = N // CHUNK

    def body(j):
        chunk = sid + NSUB * j

        @pl.when(chunk < nchunks)
        def _():
            base = chunk * CHUNK
            pltpu.sync_copy(batch.at[pl.ds(base, CHUNK)], idx_v)

            @pl.when(cid == 0)
            def _():
                pltpu.sync_copy(inva.at[pl.ds(base, CHUNK)], row_v)

            @pl.when(cid == 1)
            def _():
                pltpu.sync_copy(invb.at[pl.ds(base, CHUNK)], row_v)

            pltpu.sync_copy(row_v, acc.at[idx_v], add=True)

    pl.loop(0, (nchunks + NSUB - 1) // NSUB)(body)
    plsc.subcore_barrier()

    @pl.when(cid == 0)
    def _():
        pltpu.sync_copy(acc.at[pl.ds(sid * rows, rows)],
                        outa.at[pl.ds(sid * rows, rows)])

    @pl.when(cid == 1)
    def _():
        pltpu.sync_copy(acc.at[pl.ds(sid * rows, rows)],
                        outb.at[pl.ds(sid * rows, rows)])


@functools.cache
def _sc():
    mesh = plsc.VectorSubcoreMesh(core_axis_name="c", subcore_axis_name="s")
    cp = pltpu.CompilerParams(use_tc_tiling_on_sc=False)
    k = {}
    k["gather_pos"] = pl.kernel(
        _sc_gather_pos_body, mesh=mesh,
        out_type=[_f32((E, 8)), _f32((E, 8))],
        scratch_types=[
            pltpu.VMEM((CHUNK,), jnp.int32),
            pltpu.VMEM((CHUNK, 8), jnp.float32),
            pltpu.SemaphoreType.DMA,
        ], compiler_params=cp)
    k["gather_h"] = pl.kernel(
        _sc_gather_h_body, mesh=mesh,
        out_type=_f32((E, F)),
        scratch_types=[
            pltpu.VMEM((CHUNK,), jnp.int32),
            pltpu.VMEM((CHUNK, F), jnp.float32),
            pltpu.SemaphoreType.DMA,
        ], compiler_params=cp)
    k["scatter"] = pl.kernel(
        _sc_scatter_body, mesh=mesh,
        out_type=[_f32((NP, 24)), _f32((NP, 24))],
        scratch_types=[
            pltpu.VMEM((CHUNK,), jnp.int32),
            pltpu.VMEM((CHUNK, 24), jnp.float32),
            pltpu.VMEM((CHUNK, 24), jnp.float32),
            pltpu.VMEM_SHARED((NP, 24), jnp.float32),
            pltpu.SemaphoreType.DMA,
        ], compiler_params=cp)
    k["hist"] = pl.kernel(
        _sc_hist_body, mesh=mesh,
        out_type=[_f32((NP, 8)), _f32((G, 8))],
        scratch_types=[
            pltpu.VMEM((CHUNK,), jnp.int32),
            pltpu.VMEM((CHUNK, 8), jnp.float32),
            pltpu.VMEM_SHARED((NP, 8), jnp.float32),
            pltpu.SemaphoreType.DMA,
        ], compiler_params=cp)
    k["pool"] = pl.kernel(
        _sc_pool_body, mesh=mesh,
        out_type=[_f32((G, 32)), _f32((G, 32))],
        scratch_types=[
            pltpu.VMEM((CHUNK,), jnp.int32),
            pltpu.VMEM((CHUNK, 32), jnp.float32),
            pltpu.VMEM((CHUNK, 32), jnp.float32),
            pltpu.VMEM_SHARED((G, 32), jnp.float32),
            pltpu.SemaphoreType.DMA,
        ], compiler_params=cp)
    return k


def _elu(v):
    return jnp.where(v > 0, v, jnp.exp(jnp.minimum(v, 0.0)) - 1.0)


def _tc_embed_body(x_ref, w_ref, b_ref, o_ref):
    o_ref[...] = _elu(jnp.dot(x_ref[...], w_ref[...],
                              preferred_element_type=jnp.float32) + b_ref[...])


def _tc_embed(x, W_embed, b_embed):
    blk = 1000
    return pl.pallas_call(
        _tc_embed_body,
        grid=(N // blk,),
        in_specs=[
            pl.BlockSpec((blk, 5), lambda i: (i, 0)),
            pl.BlockSpec((5, F), lambda i: (0, 0)),
            pl.BlockSpec((1, F), lambda i: (0, 0)),
        ],
        out_specs=pl.BlockSpec((blk, F), lambda i: (i, 0)),
        out_shape=_f32((N, F)),
    )(x, W_embed, b_embed.reshape(1, F))


def _tc_gates_body(ps_ref, pd_ref, ea_ref, w1_ref, b1_ref, w2_ref, b2_ref, o_ref):
    delta = pd_ref[:, 0:3] - ps_ref[:, 0:3]
    dist = jnp.sqrt(jnp.sum(delta * delta, axis=-1, keepdims=True) + 1e-12)
    ef = jnp.concatenate([delta, dist, ea_ref[...]], axis=-1)
    for l in range(NL + 1):
        hdn = _elu(jnp.dot(ef, w1_ref[l], preferred_element_type=jnp.float32)
                   + b1_ref[l])
        ew = jnp.dot(hdn, w2_ref[l], preferred_element_type=jnp.float32) + b2_ref[l]
        o_ref[l] = jax.nn.sigmoid(ew)


def _tc_gates(possrc, posdst, edge_attr, W1s, b1s, W2s, b2s):
    blk = 1000
    return pl.pallas_call(
        _tc_gates_body,
        grid=(E // blk,),
        in_specs=[
            pl.BlockSpec((blk, 8), lambda i: (i, 0)),
            pl.BlockSpec((blk, 8), lambda i: (i, 0)),
            pl.BlockSpec((blk, 4), lambda i: (i, 0)),
            pl.BlockSpec((NL + 1, EIN, HID), lambda i: (0, 0, 0)),
            pl.BlockSpec((NL + 1, 1, HID), lambda i: (0, 0, 0)),
            pl.BlockSpec((NL + 1, HID, F), lambda i: (0, 0, 0)),
            pl.BlockSpec((NL + 1, 1, F), lambda i: (0, 0, 0)),
        ],
        out_specs=pl.BlockSpec((NL + 1, blk, F), lambda i: (0, i, 0)),
        out_shape=_f32((NL + 1, E, F)),
    )(possrc, posdst, edge_attr, W1s, b1s.reshape(NL + 1, 1, HID),
      W2s, b2s.reshape(NL + 1, 1, F))


def _tc_msg_body(g_ref, gate_ref, oa_ref, ob_ref):
    m = g_ref[...] * gate_ref[...]
    oa_ref[...] = m[:, 0:24]
    ob_ref[...] = m[:, 24:48]


def _tc_msg(g, gate):
    blk = 2000
    return pl.pallas_call(
        _tc_msg_body,
        grid=(E // blk,),
        in_specs=[
            pl.BlockSpec((blk, F), lambda i: (i, 0)),
            pl.BlockSpec((blk, F), lambda i: (i, 0)),
        ],
        out_specs=[
            pl.BlockSpec((blk, 24), lambda i: (i, 0)),
            pl.BlockSpec((blk, 24), lambda i: (i, 0)),
        ],
        out_shape=[_f32((E, 24)), _f32((E, 24))],
    )(g, gate)


def _tc_node_body(aa_ref, ab_ref, deg_ref, h_ref, w_ref, b_ref, o_ref):
    agg = jnp.concatenate([aa_ref[...], ab_ref[...]], axis=-1)
    deg = jnp.maximum(deg_ref[:, 0:1], 1.0)
    z = jnp.dot(agg / deg, w_ref[...], preferred_element_type=jnp.float32) + b_ref[...]
    o_ref[...] = h_ref[...] + _elu(z)


def _tc_node(agga, aggb, degw, h, Wn_l, bn_l):
    blk = 1000
    return pl.pallas_call(
        _tc_node_body,
        grid=(N // blk,),
        in_specs=[
            pl.BlockSpec((blk, 24), lambda i: (i, 0)),
            pl.BlockSpec((blk, 24), lambda i: (i, 0)),
            pl.BlockSpec((blk, 8), lambda i: (i, 0)),
            pl.BlockSpec((blk, F), lambda i: (i, 0)),
            pl.BlockSpec((F, F), lambda i: (0, 0)),
            pl.BlockSpec((1, F), lambda i: (0, 0)),
        ],
        out_specs=pl.BlockSpec((blk, F), lambda i: (i, 0)),
        out_shape=_f32((N, F)),
    )(agga, aggb, degw, h, Wn_l, bn_l.reshape(1, F))


def _tc_inv_body(aa_ref, ab_ref, deg_ref, w_ref, b_ref, oa_ref, ob_ref):
    agg = jnp.concatenate([aa_ref[...], ab_ref[...]], axis=-1)
    deg = jnp.maximum(deg_ref[:, 0:1], 1.0)
    z = jnp.dot(agg / deg, w_ref[...], preferred_element_type=jnp.float32) + b_ref[...]
    oa_ref[...] = z[:, 0:32]
    ob_ref[...] = z[:, 32:64]


def _tc_inv(agga, aggb, degw, Winv, binv):
    blk = 1000
    return pl.pallas_call(
        _tc_inv_body,
        grid=(N // blk,),
        in_specs=[
            pl.BlockSpec((blk, 24), lambda i: (i, 0)),
            pl.BlockSpec((blk, 24), lambda i: (i, 0)),
            pl.BlockSpec((blk, 8), lambda i: (i, 0)),
            pl.BlockSpec((F, INV), lambda i: (0, 0)),
            pl.BlockSpec((1, INV), lambda i: (0, 0)),
        ],
        out_specs=[
            pl.BlockSpec((blk, 32), lambda i: (i, 0)),
            pl.BlockSpec((blk, 32), lambda i: (i, 0)),
        ],
        out_shape=[_f32((N, 32)), _f32((N, 32))],
    )(agga, aggb, degw, Winv, binv.reshape(1, INV))


def _tc_head_body(pa_ref, pb_ref, cnt_ref, g1_ref, be1_ref, w1_ref, b1_ref,
                  g2_ref, be2_ref, w2_ref, b2_ref, o_ref):
    cnt = jnp.maximum(cnt_ref[:, 0:1], 1.0)
    pooled = jnp.concatenate([pa_ref[...], pb_ref[...]], axis=-1) / cnt

    def bnorm(z, g, b):
        m = jnp.mean(z, axis=0, keepdims=True)
        v = jnp.mean((z - m) ** 2, axis=0, keepdims=True)
        return g * (z - m) / jnp.sqrt(v + 1e-5) + b

    z = _elu(bnorm(pooled, g1_ref[...], be1_ref[...]))
    z = jnp.dot(z, w1_ref[...], preferred_element_type=jnp.float32) + b1_ref[...]
    z = _elu(bnorm(z, g2_ref[...], be2_ref[...]))
    o_ref[...] = jnp.dot(z, w2_ref[...], preferred_element_type=jnp.float32) + b2_ref[...]


def _tc_head(poola, poolb, cntw, g1, beta1, W1, b1, g2, beta2, W2, b2):
    full = lambda s: pl.BlockSpec(s, lambda: tuple(0 for _ in s))
    return pl.pallas_call(
        _tc_head_body,
        in_specs=[
            full((G, 32)), full((G, 32)), full((G, 8)),
            full((1, INV)), full((1, INV)), full((INV, INV)), full((1, INV)),
            full((1, INV)), full((1, INV)), full((INV, 1)), full((1, 1)),
        ],
        out_specs=full((G, 1)),
        out_shape=_f32((G, 1)),
    )(poola, poolb, cntw, g1.reshape(1, INV), beta1.reshape(1, INV), W1,
      b1.reshape(1, INV), g2.reshape(1, INV), beta2.reshape(1, INV), W2,
      b2.reshape(1, 1))


def kernel(x, pos, edge_index, edge_attr, batch, W_embed, b_embed, We1, be1,
           We2, be2, Wn, bn, Wie1, bie1, Wie2, bie2, Winv, binv, g1, beta1,
           W1, b1, g2, beta2, W2, b2):
    src = edge_index[0].astype(jnp.int32)
    dst = edge_index[1].astype(jnp.int32)
    batch32 = batch.astype(jnp.int32)

    pos8 = jnp.concatenate([pos, jnp.zeros((N, 5), jnp.float32)], axis=1)
    W1s = jnp.concatenate([We1, Wie1[None]], axis=0)
    b1s = jnp.concatenate([be1, bie1[None]], axis=0)
    W2s = jnp.concatenate([We2, Wie2[None]], axis=0)
    b2s = jnp.concatenate([be2, bie2[None]], axis=0)

    zfeed24 = jnp.zeros((CHUNK, 24), jnp.float32)
    zfeed32 = jnp.zeros((CHUNK, 32), jnp.float32)
    zfeed8 = jnp.zeros((CHUNK, 8), jnp.float32)
    ofeed8 = jnp.ones((CHUNK, 8), jnp.float32)

    sc = _sc()
    possrc, posdst = sc["gather_pos"](pos8, src, dst)
    gates = _tc_gates(possrc, posdst, edge_attr, W1s, b1s, W2s, b2s)
    degw, cntw = sc["hist"](dst, batch32, ofeed8, zfeed8)
    h = _tc_embed(x, W_embed, b_embed)

    for l in range(NL):
        g = sc["gather_h"](h, src)
        msga, msgb = _tc_msg(g, gates[l])
        agga, aggb = sc["scatter"](msga, msgb, dst, zfeed24)
        h = _tc_node(agga[:N], aggb[:N], degw[:N], h, Wn[l], bn[l])

    g = sc["gather_h"](h, src)
    msga, msgb = _tc_msg(g, gates[NL])
    agga, aggb = sc["scatter"](msga, msgb, dst, zfeed24)
    inva, invb = _tc_inv(agga[:N], aggb[:N], degw[:N], Winv, binv)

    poola, poolb = sc["pool"](inva, invb, batch32, zfeed32)
    return _tc_head(poola, poolb, cntw, g1, beta1, W1, b1, g2, beta2, W2, b2)


# speedup vs baseline: 1.6967x; 1.0996x over previous
import functools

import jax
import jax.numpy as jnp
from jax import lax
from jax.experimental import pallas as pl
from jax.experimental.pallas import tpu as pltpu
from jax.experimental.pallas import tpu_sc as plsc

N = 50000
E = 800000
F = 48
INV = 64
NL = 3
G = 4096
EIN = 8
HID = 64

NP = 51200
CHUNK = 400
NSUB = 16
NCORE = 2

def _f32(shape):
    return jax.ShapeDtypeStruct(shape, jnp.float32)


def _sc_gather_pos_body(pos8, src, dst, out_s, out_d, idx_v, rows_v, sem):
    cid = lax.axis_index("c")
    sid = lax.axis_index("s")
    nchunks = E // CHUNK

    def body(j):
        chunk = sid + NSUB * j

        @pl.when(chunk < nchunks)
        def _():
            base = chunk * CHUNK

            @pl.when(cid == 0)
            def _():
                pltpu.sync_copy(src.at[pl.ds(base, CHUNK)], idx_v)
                pltpu.async_copy(pos8.at[idx_v], rows_v, sem).wait()
                pltpu.sync_copy(rows_v, out_s.at[pl.ds(base, CHUNK)])

            @pl.when(cid == 1)
            def _():
                pltpu.sync_copy(dst.at[pl.ds(base, CHUNK)], idx_v)
                pltpu.async_copy(pos8.at[idx_v], rows_v, sem).wait()
                pltpu.sync_copy(rows_v, out_d.at[pl.ds(base, CHUNK)])

    pl.loop(0, nchunks // NSUB)(body)


def _sc_gather_h_body(h, src, out, idx_v, rows_v, sem):
    cid = lax.axis_index("c")
    sid = lax.axis_index("s")
    nchunks = E // CHUNK
    per_core = nchunks // NCORE

    def body(j):
        local = sid + NSUB * j

        @pl.when(local < per_core)
        def _():
            base = (cid * per_core + local) * CHUNK
            pltpu.sync_copy(src.at[pl.ds(base, CHUNK)], idx_v)
            pltpu.async_copy(h.at[idx_v], rows_v, sem).wait()
            pltpu.sync_copy(rows_v, out.at[pl.ds(base, CHUNK)])

    pl.loop(0, (per_core + NSUB - 1) // NSUB)(body)


def _sc_scatter_body(msga, msgb, dst, zfeed, outa, outb, idx_v, msg_v, z_v, acc, sem):
    cid = lax.axis_index("c")
    sid = lax.axis_index("s")

    pltpu.sync_copy(zfeed, z_v)

    def zbody(j):
        chunk = sid + NSUB * j
        pltpu.sync_copy(z_v, acc.at[pl.ds(chunk * CHUNK, CHUNK)])

    pl.loop(0, NP // CHUNK // NSUB)(zbody)
    plsc.subcore_barrier()

    nchunks = E // CHUNK

    def body(j):
        chunk = sid + NSUB * j

        @pl.when(chunk < nchunks)
        def _():
            base = chunk * CHUNK
            pltpu.sync_copy(dst.at[pl.ds(base, CHUNK)], idx_v)

            @pl.when(cid == 0)
            def _():
                pltpu.sync_copy(msga.at[pl.ds(base, CHUNK)], msg_v)

            @pl.when(cid == 1)
            def _():
                pltpu.sync_copy(msgb.at[pl.ds(base, CHUNK)], msg_v)

            pltpu.sync_copy(msg_v, acc.at[idx_v], add=True)

    pl.loop(0, nchunks // NSUB)(body)
    plsc.subcore_barrier()

    rows = NP // NSUB
    base = sid * rows

    @pl.when(cid == 0)
    def _():
        pltpu.sync_copy(acc.at[pl.ds(base, rows)], outa.at[pl.ds(base, rows)])

    @pl.when(cid == 1)
    def _():
        pltpu.sync_copy(acc.at[pl.ds(base, rows)], outb.at[pl.ds(base, rows)])


def _sc_hist_body(dst, batch, ofeed, zfeed8, deg_out, cnt_out, idx_v, ones_v, acc, sem):
    cid = lax.axis_index("c")
    sid = lax.axis_index("s")

    pltpu.sync_copy(zfeed8, ones_v)

    def zbody(j):
        chunk = sid + NSUB * j

        @pl.when(cid == 0)
        def _():
            pltpu.sync_copy(ones_v, acc.at[pl.ds(chunk * CHUNK, CHUNK)])

        @pl.when(jnp.logical_and(cid == 1, chunk < G // CHUNK + 1))
        def _():
            @pl.when(chunk * CHUNK < G - CHUNK + 1)
            def _():
                pltpu.sync_copy(ones_v, acc.at[pl.ds(chunk * CHUNK, CHUNK)])

            @pl.when(chunk == G // CHUNK)
            def _():
                pltpu.sync_copy(ones_v.at[pl.ds(0, G % CHUNK)],
                                acc.at[pl.ds(G - (G % CHUNK), G % CHUNK)])

    pl.loop(0, NP // CHUNK // NSUB)(zbody)
    pltpu.sync_copy(ofeed, ones_v)
    plsc.subcore_barrier()

    ne = E // CHUNK
    nn = N // CHUNK

    def body(j):
        chunk = sid + NSUB * j

        @pl.when(cid == 0)
        def _():
            pltpu.sync_copy(dst.at[pl.ds(chunk * CHUNK, CHUNK)], idx_v)
            pltpu.sync_copy(ones_v, acc.at[idx_v], add=True)

        @pl.when(jnp.logical_and(cid == 1, chunk < nn))
        def _():
            pltpu.sync_copy(batch.at[pl.ds(chunk * CHUNK, CHUNK)], idx_v)
            pltpu.sync_copy(ones_v, acc.at[idx_v], add=True)

    pl.loop(0, ne // NSUB)(body)
    plsc.subcore_barrier()

    @pl.when(cid == 0)
    def _():
        rows = NP // NSUB
        pltpu.sync_copy(acc.at[pl.ds(sid * rows, rows)],
                        deg_out.at[pl.ds(sid * rows, rows)])

    @pl.when(cid == 1)
    def _():
        rows = G // NSUB
        pltpu.sync_copy(acc.at[pl.ds(sid * rows, rows)],
                        cnt_out.at[pl.ds(sid * rows, rows)])


def _sc_pool_body(inva, invb, batch, zfeed, outa, outb, idx_v, row_v, z_v, acc, sem):
    cid = lax.axis_index("c")
    sid = lax.axis_index("s")

    pltpu.sync_copy(zfeed, z_v)
    rows = G // NSUB
    pltpu.sync_copy(z_v.at[pl.ds(0, rows)], acc.at[pl.ds(sid * rows, rows)])
    plsc.subcore_barrier()

    nchunks = N // CHUNK

    def body(j):
        chunk = sid + NSUB * j

        @pl.when(chunk < nchunks)
        def _():
            base = chunk * CHUNK
            pltpu.sync_copy(batch.at[pl.ds(base, CHUNK)], idx_v)

            @pl.when(cid == 0)
            def _():
                pltpu.sync_copy(inva.at[pl.ds(base, CHUNK)], row_v)

            @pl.when(cid == 1)
            def _():
                pltpu.sync_copy(invb.at[pl.ds(base, CHUNK)], row_v)

            pltpu.sync_copy(row_v, acc.at[idx_v], add=True)

    pl.loop(0, (nchunks + NSUB - 1) // NSUB)(body)
    plsc.subcore_barrier()

    @pl.when(cid == 0)
    def _():
        pltpu.sync_copy(acc.at[pl.ds(sid * rows, rows)],
                        outa.at[pl.ds(sid * rows, rows)])

    @pl.when(cid == 1)
    def _():
        pltpu.sync_copy(acc.at[pl.ds(sid * rows, rows)],
                        outb.at[pl.ds(sid * rows, rows)])


@functools.cache
def _sc():
    mesh = plsc.VectorSubcoreMesh(core_axis_name="c", subcore_axis_name="s")
    cp = pltpu.CompilerParams(use_tc_tiling_on_sc=False)
    k = {}
    k["gather_pos"] = pl.kernel(
        _sc_gather_pos_body, mesh=mesh,
        out_type=[_f32((E, 8)), _f32((E, 8))],
        scratch_types=[
            pltpu.VMEM((CHUNK,), jnp.int32),
            pltpu.VMEM((CHUNK, 8), jnp.float32),
            pltpu.SemaphoreType.DMA,
        ], compiler_params=cp)
    k["gather_h"] = pl.kernel(
        _sc_gather_h_body, mesh=mesh,
        out_type=_f32((E, F)),
        scratch_types=[
            pltpu.VMEM((CHUNK,), jnp.int32),
            pltpu.VMEM((CHUNK, F), jnp.float32),
            pltpu.SemaphoreType.DMA,
        ], compiler_params=cp)
    k["scatter"] = pl.kernel(
        _sc_scatter_body, mesh=mesh,
        out_type=[_f32((NP, 24)), _f32((NP, 24))],
        scratch_types=[
            pltpu.VMEM((CHUNK,), jnp.int32),
            pltpu.VMEM((CHUNK, 24), jnp.float32),
            pltpu.VMEM((CHUNK, 24), jnp.float32),
            pltpu.VMEM_SHARED((NP, 24), jnp.float32),
            pltpu.SemaphoreType.DMA,
        ], compiler_params=cp)
    k["hist"] = pl.kernel(
        _sc_hist_body, mesh=mesh,
        out_type=[_f32((NP, 8)), _f32((G, 8))],
        scratch_types=[
            pltpu.VMEM((CHUNK,), jnp.int32),
            pltpu.VMEM((CHUNK, 8), jnp.float32),
            pltpu.VMEM_SHARED((NP, 8), jnp.float32),
            pltpu.SemaphoreType.DMA,
        ], compiler_params=cp)
    k["pool"] = pl.kernel(
        _sc_pool_body, mesh=mesh,
        out_type=[_f32((G, 32)), _f32((G, 32))],
        scratch_types=[
            pltpu.VMEM((CHUNK,), jnp.int32),
            pltpu.VMEM((CHUNK, 32), jnp.float32),
            pltpu.VMEM((CHUNK, 32), jnp.float32),
            pltpu.VMEM_SHARED((G, 32), jnp.float32),
            pltpu.SemaphoreType.DMA,
        ], compiler_params=cp)
    return k


def _elu(v):
    return jnp.where(v > 0, v, jnp.exp(jnp.minimum(v, 0.0)) - 1.0)


def _tc_embed_body(x_ref, w_ref, b_ref, o_ref):
    o_ref[...] = _elu(jnp.dot(x_ref[...], w_ref[...],
                              preferred_element_type=jnp.float32) + b_ref[...])


def _tc_embed(x, W_embed, b_embed):
    blk = 1000
    return pl.pallas_call(
        _tc_embed_body,
        grid=(N // blk,),
        in_specs=[
            pl.BlockSpec((blk, 5), lambda i: (i, 0)),
            pl.BlockSpec((5, F), lambda i: (0, 0)),
            pl.BlockSpec((1, F), lambda i: (0, 0)),
        ],
        out_specs=pl.BlockSpec((blk, F), lambda i: (i, 0)),
        out_shape=_f32((N, F)),
    )(x, W_embed, b_embed.reshape(1, F))


def _tc_gates_body(ps_ref, pd_ref, ea_ref, w1_ref, b1_ref, w2_ref, b2_ref, o_ref):
    delta = pd_ref[:, 0:3] - ps_ref[:, 0:3]
    dist = jnp.sqrt(jnp.sum(delta * delta, axis=-1, keepdims=True) + 1e-12)
    ef = jnp.concatenate([delta, dist, ea_ref[...]], axis=-1)
    for l in range(NL + 1):
        hdn = _elu(jnp.dot(ef, w1_ref[l], preferred_element_type=jnp.float32)
                   + b1_ref[l])
        ew = jnp.dot(hdn, w2_ref[l], preferred_element_type=jnp.float32) + b2_ref[l]
        o_ref[l] = jax.nn.sigmoid(ew)


def _tc_gates(possrc, posdst, edge_attr, W1s, b1s, W2s, b2s):
    blk = 1000
    return pl.pallas_call(
        _tc_gates_body,
        grid=(E // blk,),
        in_specs=[
            pl.BlockSpec((blk, 8), lambda i: (i, 0)),
            pl.BlockSpec((blk, 8), lambda i: (i, 0)),
            pl.BlockSpec((blk, 4), lambda i: (i, 0)),
            pl.BlockSpec((NL + 1, EIN, HID), lambda i: (0, 0, 0)),
            pl.BlockSpec((NL + 1, 1, HID), lambda i: (0, 0, 0)),
            pl.BlockSpec((NL + 1, HID, F), lambda i: (0, 0, 0)),
            pl.BlockSpec((NL + 1, 1, F), lambda i: (0, 0, 0)),
        ],
        out_specs=pl.BlockSpec((NL + 1, blk, F), lambda i: (0, i, 0)),
        out_shape=_f32((NL + 1, E, F)),
    )(possrc, posdst, edge_attr, W1s, b1s.reshape(NL + 1, 1, HID),
      W2s, b2s.reshape(NL + 1, 1, F))


def _tc_msg_body(g_ref, gate_ref, oa_ref, ob_ref):
    m = g_ref[...] * gate_ref[0]
    oa_ref[...] = m[:, 0:24]
    ob_ref[...] = m[:, 24:48]


def _tc_msg(g, gates, l):
    blk = 2000
    return pl.pallas_call(
        _tc_msg_body,
        grid=(E // blk,),
        in_specs=[
            pl.BlockSpec((blk, F), lambda i: (i, 0)),
            pl.BlockSpec((1, blk, F), lambda i, _l=l: (_l, i, 0)),
        ],
        out_specs=[
            pl.BlockSpec((blk, 24), lambda i: (i, 0)),
            pl.BlockSpec((blk, 24), lambda i: (i, 0)),
        ],
        out_shape=[_f32((E, 24)), _f32((E, 24))],
    )(g, gates)


def _tc_node_body(aa_ref, ab_ref, deg_ref, h_ref, w_ref, b_ref, o_ref):
    agg = jnp.concatenate([aa_ref[...], ab_ref[...]], axis=-1)
    deg = jnp.maximum(deg_ref[:, 0:1], 1.0)
    z = jnp.dot(agg / deg, w_ref[...], preferred_element_type=jnp.float32) + b_ref[...]
    o_ref[...] = h_ref[...] + _elu(z)


def _tc_node(agga, aggb, degw, h, Wn_l, bn_l):
    blk = 1000
    return pl.pallas_call(
        _tc_node_body,
        grid=(N // blk,),
        in_specs=[
            pl.BlockSpec((blk, 24), lambda i: (i, 0)),
            pl.BlockSpec((blk, 24), lambda i: (i, 0)),
            pl.BlockSpec((blk, 8), lambda i: (i, 0)),
            pl.BlockSpec((blk, F), lambda i: (i, 0)),
            pl.BlockSpec((F, F), lambda i: (0, 0)),
            pl.BlockSpec((1, F), lambda i: (0, 0)),
        ],
        out_specs=pl.BlockSpec((blk, F), lambda i: (i, 0)),
        out_shape=_f32((N, F)),
    )(agga, aggb, degw, h, Wn_l, bn_l.reshape(1, F))


def _tc_inv_body(aa_ref, ab_ref, deg_ref, w_ref, b_ref, oa_ref, ob_ref):
    agg = jnp.concatenate([aa_ref[...], ab_ref[...]], axis=-1)
    deg = jnp.maximum(deg_ref[:, 0:1], 1.0)
    z = jnp.dot(agg / deg, w_ref[...], preferred_element_type=jnp.float32) + b_ref[...]
    oa_ref[...] = z[:, 0:32]
    ob_ref[...] = z[:, 32:64]


def _tc_inv(agga, aggb, degw, Winv, binv):
    blk = 1000
    return pl.pallas_call(
        _tc_inv_body,
        grid=(N // blk,),
        in_specs=[
            pl.BlockSpec((blk, 24), lambda i: (i, 0)),
            pl.BlockSpec((blk, 24), lambda i: (i, 0)),
            pl.BlockSpec((blk, 8), lambda i: (i, 0)),
            pl.BlockSpec((F, INV), lambda i: (0, 0)),
            pl.BlockSpec((1, INV), lambda i: (0, 0)),
        ],
        out_specs=[
            pl.BlockSpec((blk, 32), lambda i: (i, 0)),
            pl.BlockSpec((blk, 32), lambda i: (i, 0)),
        ],
        out_shape=[_f32((N, 32)), _f32((N, 32))],
    )(agga, aggb, degw, Winv, binv.reshape(1, INV))


def _tc_head_body(pa_ref, pb_ref, cnt_ref, g1_ref, be1_ref, w1_ref, b1_ref,
                  g2_ref, be2_ref, w2_ref, b2_ref, o_ref):
    cnt = jnp.maximum(cnt_ref[:, 0:1], 1.0)
    pooled = jnp.concatenate([pa_ref[...], pb_ref[...]], axis=-1) / cnt

    def bnorm(z, g, b):
        m = jnp.mean(z, axis=0, keepdims=True)
        v = jnp.mean((z - m) ** 2, axis=0, keepdims=True)
        return g * (z - m) / jnp.sqrt(v + 1e-5) + b

    z = _elu(bnorm(pooled, g1_ref[...], be1_ref[...]))
    z = jnp.dot(z, w1_ref[...], preferred_element_type=jnp.float32) + b1_ref[...]
    z = _elu(bnorm(z, g2_ref[...], be2_ref[...]))
    o_ref[...] = jnp.dot(z, w2_ref[...], preferred_element_type=jnp.float32) + b2_ref[...]


def _tc_head(poola, poolb, cntw, g1, beta1, W1, b1, g2, beta2, W2, b2):
    full = lambda s: pl.BlockSpec(s, lambda: tuple(0 for _ in s))
    return pl.pallas_call(
        _tc_head_body,
        in_specs=[
            full((G, 32)), full((G, 32)), full((G, 8)),
            full((1, INV)), full((1, INV)), full((INV, INV)), full((1, INV)),
            full((1, INV)), full((1, INV)), full((INV, 1)), full((1, 1)),
        ],
        out_specs=full((G, 1)),
        out_shape=_f32((G, 1)),
    )(poola, poolb, cntw, g1.reshape(1, INV), beta1.reshape(1, INV), W1,
      b1.reshape(1, INV), g2.reshape(1, INV), beta2.reshape(1, INV), W2,
      b2.reshape(1, 1))


def kernel(x, pos, edge_index, edge_attr, batch, W_embed, b_embed, We1, be1,
           We2, be2, Wn, bn, Wie1, bie1, Wie2, bie2, Winv, binv, g1, beta1,
           W1, b1, g2, beta2, W2, b2):
    src = edge_index[0].astype(jnp.int32)
    dst = edge_index[1].astype(jnp.int32)
    batch32 = batch.astype(jnp.int32)

    pos8 = jnp.concatenate([pos, jnp.zeros((N, 5), jnp.float32)], axis=1)
    W1s = jnp.concatenate([We1, Wie1[None]], axis=0)
    b1s = jnp.concatenate([be1, bie1[None]], axis=0)
    W2s = jnp.concatenate([We2, Wie2[None]], axis=0)
    b2s = jnp.concatenate([be2, bie2[None]], axis=0)

    zfeed24 = jnp.zeros((CHUNK, 24), jnp.float32)
    zfeed32 = jnp.zeros((CHUNK, 32), jnp.float32)
    zfeed8 = jnp.zeros((CHUNK, 8), jnp.float32)
    ofeed8 = jnp.ones((CHUNK, 8), jnp.float32)

    sc = _sc()
    possrc, posdst = sc["gather_pos"](pos8, src, dst)
    gates = _tc_gates(possrc, posdst, edge_attr, W1s, b1s, W2s, b2s)
    degw, cntw = sc["hist"](dst, batch32, ofeed8, zfeed8)
    h = _tc_embed(x, W_embed, b_embed)

    for l in range(NL):
        g = sc["gather_h"](h, src)
        msga, msgb = _tc_msg(g, gates, l)
        agga, aggb = sc["scatter"](msga, msgb, dst, zfeed24)
        h = _tc_node(agga, aggb, degw, h, Wn[l], bn[l])

    g = sc["gather_h"](h, src)
    msga, msgb = _tc_msg(g, gates, NL)
    agga, aggb = sc["scatter"](msga, msgb, dst, zfeed24)
    inva, invb = _tc_inv(agga, aggb, degw, Winv, binv)

    poola, poolb = sc["pool"](inva, invb, batch32, zfeed32)
    return _tc_head(poola, poolb, cntw, g1, beta1, W1, b1, g2, beta2, W2, b2)


# speedup vs baseline: 1.8353x; 1.0817x over previous
import functools

import jax
import jax.numpy as jnp
from jax import lax
from jax.experimental import pallas as pl
from jax.experimental.pallas import tpu as pltpu
from jax.experimental.pallas import tpu_sc as plsc

N = 50000
E = 800000
F = 48
INV = 64
NL = 3
G = 4096
EIN = 8
HID = 64

NP = 51200
CHUNK = 200
NSUB = 16
NCORE = 2

def _f32(shape):
    return jax.ShapeDtypeStruct(shape, jnp.float32)


def _sc_gather_pos_body(pos8, src, dst, out_s, out_d, idx_v, rows_v, sem):
    cid = lax.axis_index("c")
    sid = lax.axis_index("s")
    nchunks = E // CHUNK

    def body(j):
        chunk = sid + NSUB * j

        @pl.when(chunk < nchunks)
        def _():
            base = chunk * CHUNK

            @pl.when(cid == 0)
            def _():
                pltpu.sync_copy(src.at[pl.ds(base, CHUNK)], idx_v)
                pltpu.async_copy(pos8.at[idx_v], rows_v, sem).wait()
                pltpu.sync_copy(rows_v, out_s.at[pl.ds(base, CHUNK)])

            @pl.when(cid == 1)
            def _():
                pltpu.sync_copy(dst.at[pl.ds(base, CHUNK)], idx_v)
                pltpu.async_copy(pos8.at[idx_v], rows_v, sem).wait()
                pltpu.sync_copy(rows_v, out_d.at[pl.ds(base, CHUNK)])

    pl.loop(0, nchunks // NSUB)(body)


def _sc_fused_body_maker(l):
    def body(h32, h16, gates, src, dst, zfeed, outa, outb,
             srcv, dstv, hv32, hv16, gv, msgv, acc, sem):
        cid = lax.axis_index("c")
        sid = lax.axis_index("s")

        pltpu.sync_copy(zfeed.at[pl.ds(0, CHUNK)], msgv)

        def zbody(j):
            chunk = sid + NSUB * j
            pltpu.sync_copy(zfeed.at[pl.ds(0, CHUNK)],
                            acc.at[pl.ds(chunk * CHUNK, CHUNK)])

        pl.loop(0, NP // CHUNK // NSUB)(zbody)
        plsc.subcore_barrier()

        nchunks = E // CHUNK

        def body_j(j):
            chunk = sid + NSUB * j
            base = chunk * CHUNK
            pltpu.sync_copy(src.at[pl.ds(base, CHUNK)], srcv)
            pltpu.sync_copy(dst.at[pl.ds(base, CHUNK)], dstv)
            pltpu.sync_copy(gates.at[l, pl.ds(base, CHUNK)], gv)

            @pl.when(cid == 0)
            def _():
                pltpu.async_copy(h32.at[srcv], hv32, sem).wait()

                def mul0(k):
                    msgv[k, pl.ds(0, 16)] = hv32[k, pl.ds(0, 16)] * gv[k, pl.ds(0, 16)]
                    msgv[k, pl.ds(16, 16)] = hv32[k, pl.ds(16, 16)] * gv[k, pl.ds(16, 16)]

                pl.loop(0, CHUNK, unroll=8)(mul0)

            @pl.when(cid == 1)
            def _():
                pltpu.async_copy(h16.at[srcv], hv16, sem).wait()

                def mul1(k):
                    msgv[k, pl.ds(0, 16)] = hv16[k, pl.ds(0, 16)] * gv[k, pl.ds(32, 16)]

                pl.loop(0, CHUNK, unroll=8)(mul1)

            pltpu.sync_copy(msgv, acc.at[dstv], add=True)

        pl.loop(0, nchunks // NSUB)(body_j)
        plsc.subcore_barrier()

        rows = NP // NSUB
        rbase = sid * rows

        @pl.when(cid == 0)
        def _():
            pltpu.sync_copy(acc.at[pl.ds(rbase, rows)], outa.at[pl.ds(rbase, rows)])

        @pl.when(cid == 1)
        def _():
            pltpu.sync_copy(acc.at[pl.ds(rbase, rows)], outb.at[pl.ds(rbase, rows)])

    return body


def _sc_hist_body(dst, batch, ofeed, zfeed8, deg_out, cnt_out, idx_v, ones_v, acc, sem):
    cid = lax.axis_index("c")
    sid = lax.axis_index("s")

    def zbody(j):
        chunk = sid + NSUB * j

        @pl.when(cid == 0)
        def _():
            pltpu.sync_copy(zfeed8.at[pl.ds(0, CHUNK)],
                            acc.at[pl.ds(chunk * CHUNK, CHUNK)])

        @pl.when(jnp.logical_and(cid == 1, chunk < G // CHUNK + 1))
        def _():
            @pl.when(chunk * CHUNK < G - CHUNK + 1)
            def _():
                pltpu.sync_copy(zfeed8.at[pl.ds(0, CHUNK)],
                                acc.at[pl.ds(chunk * CHUNK, CHUNK)])

            @pl.when(chunk == G // CHUNK)
            def _():
                pltpu.sync_copy(zfeed8.at[pl.ds(0, G % CHUNK)],
                                acc.at[pl.ds(G - (G % CHUNK), G % CHUNK)])

    pl.loop(0, NP // CHUNK // NSUB)(zbody)
    pltpu.sync_copy(ofeed, ones_v)
    plsc.subcore_barrier()

    ne = E // CHUNK
    nn = N // CHUNK

    def body(j):
        chunk = sid + NSUB * j

        @pl.when(cid == 0)
        def _():
            pltpu.sync_copy(dst.at[pl.ds(chunk * CHUNK, CHUNK)], idx_v)
            pltpu.sync_copy(ones_v, acc.at[idx_v], add=True)

        @pl.when(jnp.logical_and(cid == 1, chunk < nn))
        def _():
            pltpu.sync_copy(batch.at[pl.ds(chunk * CHUNK, CHUNK)], idx_v)
            pltpu.sync_copy(ones_v, acc.at[idx_v], add=True)

    pl.loop(0, ne // NSUB)(body)
    plsc.subcore_barrier()

    @pl.when(cid == 0)
    def _():
        rows = NP // NSUB
        pltpu.sync_copy(acc.at[pl.ds(sid * rows, rows)],
                        deg_out.at[pl.ds(sid * rows, rows)])

    @pl.when(cid == 1)
    def _():
        rows = G // NSUB
        pltpu.sync_copy(acc.at[pl.ds(sid * rows, rows)],
                        cnt_out.at[pl.ds(sid * rows, rows)])


def _sc_pool_body(inva, invb, batch, zfeed, outa, outb, idx_v, row_v, acc, sem):
    cid = lax.axis_index("c")
    sid = lax.axis_index("s")

    rows = G // NSUB
    pltpu.sync_copy(zfeed.at[pl.ds(0, rows)], acc.at[pl.ds(sid * rows, rows)])
    plsc.subcore_barrier()

    nchunks = N // CHUNK

    def body(j):
        chunk = sid + NSUB * j

        @pl.when(chunk < nchunks)
        def _():
            base = chunk * CHUNK
            pltpu.sync_copy(batch.at[pl.ds(base, CHUNK)], idx_v)

            @pl.when(cid == 0)
            def _():
                pltpu.sync_copy(inva.at[pl.ds(base, CHUNK)], row_v)

            @pl.when(cid == 1)
            def _():
                pltpu.sync_copy(invb.at[pl.ds(base, CHUNK)], row_v)

            pltpu.sync_copy(row_v, acc.at[idx_v], add=True)

    pl.loop(0, (nchunks + NSUB - 1) // NSUB)(body)
    plsc.subcore_barrier()

    @pl.when(cid == 0)
    def _():
        pltpu.sync_copy(acc.at[pl.ds(sid * rows, rows)],
                        outa.at[pl.ds(sid * rows, rows)])

    @pl.when(cid == 1)
    def _():
        pltpu.sync_copy(acc.at[pl.ds(sid * rows, rows)],
                        outb.at[pl.ds(sid * rows, rows)])


@functools.cache
def _sc():
    mesh = plsc.VectorSubcoreMesh(core_axis_name="c", subcore_axis_name="s")
    cp = pltpu.CompilerParams(use_tc_tiling_on_sc=False)
    k = {}
    k["gather_pos"] = pl.kernel(
        _sc_gather_pos_body, mesh=mesh,
        out_type=[_f32((E, 8)), _f32((E, 8))],
        scratch_types=[
            pltpu.VMEM((CHUNK,), jnp.int32),
            pltpu.VMEM((CHUNK, 8), jnp.float32),
            pltpu.SemaphoreType.DMA,
        ], compiler_params=cp)
    for l in range(NL + 1):
        k[f"fused{l}"] = pl.kernel(
            _sc_fused_body_maker(l), mesh=mesh,
            out_type=[_f32((NP, 32)), _f32((NP, 32))],
            scratch_types=[
                pltpu.VMEM((CHUNK,), jnp.int32),
                pltpu.VMEM((CHUNK,), jnp.int32),
                pltpu.VMEM((CHUNK, 32), jnp.float32),
                pltpu.VMEM((CHUNK, 16), jnp.float32),
                pltpu.VMEM((CHUNK, F), jnp.float32),
                pltpu.VMEM((CHUNK, 32), jnp.float32),
                pltpu.VMEM_SHARED((NP, 32), jnp.float32),
                pltpu.SemaphoreType.DMA,
            ], compiler_params=cp)
    k["hist"] = pl.kernel(
        _sc_hist_body, mesh=mesh,
        out_type=[_f32((NP, 8)), _f32((G, 8))],
        scratch_types=[
            pltpu.VMEM((CHUNK,), jnp.int32),
            pltpu.VMEM((CHUNK, 8), jnp.float32),
            pltpu.VMEM_SHARED((NP, 8), jnp.float32),
            pltpu.SemaphoreType.DMA,
        ], compiler_params=cp)
    k["pool"] = pl.kernel(
        _sc_pool_body, mesh=mesh,
        out_type=[_f32((G, 32)), _f32((G, 32))],
        scratch_types=[
            pltpu.VMEM((CHUNK,), jnp.int32),
            pltpu.VMEM((CHUNK, 32), jnp.float32),
            pltpu.VMEM_SHARED((G, 32), jnp.float32),
            pltpu.SemaphoreType.DMA,
        ], compiler_params=cp)
    return k


def _elu(v):
    return jnp.where(v > 0, v, jnp.exp(jnp.minimum(v, 0.0)) - 1.0)


def _tc_embed_body(x_ref, w_ref, b_ref, o32_ref, o16_ref):
    h = _elu(jnp.dot(x_ref[...], w_ref[...],
                     preferred_element_type=jnp.float32) + b_ref[...])
    o32_ref[...] = h[:, 0:32]
    o16_ref[...] = h[:, 32:48]


def _tc_embed(x, W_embed, b_embed):
    blk = 1000
    return pl.pallas_call(
        _tc_embed_body,
        grid=(N // blk,),
        in_specs=[
            pl.BlockSpec((blk, 5), lambda i: (i, 0)),
            pl.BlockSpec((5, F), lambda i: (0, 0)),
            pl.BlockSpec((1, F), lambda i: (0, 0)),
        ],
        out_specs=[
            pl.BlockSpec((blk, 32), lambda i: (i, 0)),
            pl.BlockSpec((blk, 16), lambda i: (i, 0)),
        ],
        out_shape=[_f32((N, 32)), _f32((N, 16))],
    )(x, W_embed, b_embed.reshape(1, F))


def _tc_gates_body(ps_ref, pd_ref, ea_ref, w1_ref, b1_ref, w2_ref, b2_ref, o_ref):
    delta = pd_ref[:, 0:3] - ps_ref[:, 0:3]
    dist = jnp.sqrt(jnp.sum(delta * delta, axis=-1, keepdims=True) + 1e-12)
    ef = jnp.concatenate([delta, dist, ea_ref[...]], axis=-1)
    for l in range(NL + 1):
        hdn = _elu(jnp.dot(ef, w1_ref[l], preferred_element_type=jnp.float32)
                   + b1_ref[l])
        ew = jnp.dot(hdn, w2_ref[l], preferred_element_type=jnp.float32) + b2_ref[l]
        o_ref[l] = jax.nn.sigmoid(ew)


def _tc_gates(possrc, posdst, edge_attr, W1s, b1s, W2s, b2s):
    blk = 1000
    return pl.pallas_call(
        _tc_gates_body,
        grid=(E // blk,),
        in_specs=[
            pl.BlockSpec((blk, 8), lambda i: (i, 0)),
            pl.BlockSpec((blk, 8), lambda i: (i, 0)),
            pl.BlockSpec((blk, 4), lambda i: (i, 0)),
            pl.BlockSpec((NL + 1, EIN, HID), lambda i: (0, 0, 0)),
            pl.BlockSpec((NL + 1, 1, HID), lambda i: (0, 0, 0)),
            pl.BlockSpec((NL + 1, HID, F), lambda i: (0, 0, 0)),
            pl.BlockSpec((NL + 1, 1, F), lambda i: (0, 0, 0)),
        ],
        out_specs=pl.BlockSpec((NL + 1, blk, F), lambda i: (0, i, 0)),
        out_shape=_f32((NL + 1, E, F)),
    )(possrc, posdst, edge_attr, W1s, b1s.reshape(NL + 1, 1, HID),
      W2s, b2s.reshape(NL + 1, 1, F))


def _tc_node_body(aa_ref, ab_ref, deg_ref, h32_ref, h16_ref, w_ref, b_ref,
                  o32_ref, o16_ref):
    agg = jnp.concatenate([aa_ref[...], ab_ref[:, 0:16]], axis=-1)
    deg = jnp.maximum(deg_ref[:, 0:1], 1.0)
    h = jnp.concatenate([h32_ref[...], h16_ref[...]], axis=-1)
    z = jnp.dot(agg / deg, w_ref[...], preferred_element_type=jnp.float32) + b_ref[...]
    hn = h + _elu(z)
    o32_ref[...] = hn[:, 0:32]
    o16_ref[...] = hn[:, 32:48]


def _tc_node(agga, aggb, degw, h32, h16, Wn_l, bn_l):
    blk = 1000
    return pl.pallas_call(
        _tc_node_body,
        grid=(N // blk,),
        in_specs=[
            pl.BlockSpec((blk, 32), lambda i: (i, 0)),
            pl.BlockSpec((blk, 32), lambda i: (i, 0)),
            pl.BlockSpec((blk, 8), lambda i: (i, 0)),
            pl.BlockSpec((blk, 32), lambda i: (i, 0)),
            pl.BlockSpec((blk, 16), lambda i: (i, 0)),
            pl.BlockSpec((F, F), lambda i: (0, 0)),
            pl.BlockSpec((1, F), lambda i: (0, 0)),
        ],
        out_specs=[
            pl.BlockSpec((blk, 32), lambda i: (i, 0)),
            pl.BlockSpec((blk, 16), lambda i: (i, 0)),
        ],
        out_shape=[_f32((N, 32)), _f32((N, 16))],
    )(agga, aggb, degw, h32, h16, Wn_l, bn_l.reshape(1, F))


def _tc_inv_body(aa_ref, ab_ref, deg_ref, w_ref, b_ref, oa_ref, ob_ref):
    agg = jnp.concatenate([aa_ref[...], ab_ref[:, 0:16]], axis=-1)
    deg = jnp.maximum(deg_ref[:, 0:1], 1.0)
    z = jnp.dot(agg / deg, w_ref[...], preferred_element_type=jnp.float32) + b_ref[...]
    oa_ref[...] = z[:, 0:32]
    ob_ref[...] = z[:, 32:64]


def _tc_inv(agga, aggb, degw, Winv, binv):
    blk = 1000
    return pl.pallas_call(
        _tc_inv_body,
        grid=(N // blk,),
        in_specs=[
            pl.BlockSpec((blk, 32), lambda i: (i, 0)),
            pl.BlockSpec((blk, 32), lambda i: (i, 0)),
            pl.BlockSpec((blk, 8), lambda i: (i, 0)),
            pl.BlockSpec((F, INV), lambda i: (0, 0)),
            pl.BlockSpec((1, INV), lambda i: (0, 0)),
        ],
        out_specs=[
            pl.BlockSpec((blk, 32), lambda i: (i, 0)),
            pl.BlockSpec((blk, 32), lambda i: (i, 0)),
        ],
        out_shape=[_f32((N, 32)), _f32((N, 32))],
    )(agga, aggb, degw, Winv, binv.reshape(1, INV))


def _tc_head_body(pa_ref, pb_ref, cnt_ref, g1_ref, be1_ref, w1_ref, b1_ref,
                  g2_ref, be2_ref, w2_ref, b2_ref, o_ref):
    cnt = jnp.maximum(cnt_ref[:, 0:1], 1.0)
    pooled = jnp.concatenate([pa_ref[...], pb_ref[...]], axis=-1) / cnt

    def bnorm(z, g, b):
        m = jnp.mean(z, axis=0, keepdims=True)
        v = jnp.mean((z - m) ** 2, axis=0, keepdims=True)
        return g * (z - m) / jnp.sqrt(v + 1e-5) + b

    z = _elu(bnorm(pooled, g1_ref[...], be1_ref[...]))
    z = jnp.dot(z, w1_ref[...], preferred_element_type=jnp.float32) + b1_ref[...]
    z = _elu(bnorm(z, g2_ref[...], be2_ref[...]))
    o_ref[...] = jnp.dot(z, w2_ref[...], preferred_element_type=jnp.float32) + b2_ref[...]


def _tc_head(poola, poolb, cntw, g1, beta1, W1, b1, g2, beta2, W2, b2):
    full = lambda s: pl.BlockSpec(s, lambda: tuple(0 for _ in s))
    return pl.pallas_call(
        _tc_head_body,
        in_specs=[
            full((G, 32)), full((G, 32)), full((G, 8)),
            full((1, INV)), full((1, INV)), full((INV, INV)), full((1, INV)),
            full((1, INV)), full((1, INV)), full((INV, 1)), full((1, 1)),
        ],
        out_specs=full((G, 1)),
        out_shape=_f32((G, 1)),
    )(poola, poolb, cntw, g1.reshape(1, INV), beta1.reshape(1, INV), W1,
      b1.reshape(1, INV), g2.reshape(1, INV), beta2.reshape(1, INV), W2,
      b2.reshape(1, 1))


def kernel(x, pos, edge_index, edge_attr, batch, W_embed, b_embed, We1, be1,
           We2, be2, Wn, bn, Wie1, bie1, Wie2, bie2, Winv, binv, g1, beta1,
           W1, b1, g2, beta2, W2, b2):
    src = edge_index[0].astype(jnp.int32)
    dst = edge_index[1].astype(jnp.int32)
    batch32 = batch.astype(jnp.int32)

    pos8 = jnp.concatenate([pos, jnp.zeros((N, 5), jnp.float32)], axis=1)
    W1s = jnp.concatenate([We1, Wie1[None]], axis=0)
    b1s = jnp.concatenate([be1, bie1[None]], axis=0)
    W2s = jnp.concatenate([We2, Wie2[None]], axis=0)
    b2s = jnp.concatenate([be2, bie2[None]], axis=0)

    zfeed32 = jnp.zeros((400, 32), jnp.float32)
    zfeed8 = jnp.zeros((400, 8), jnp.float32)
    ofeed8 = jnp.ones((CHUNK, 8), jnp.float32)

    sc = _sc()
    possrc, posdst = sc["gather_pos"](pos8, src, dst)
    gates = _tc_gates(possrc, posdst, edge_attr, W1s, b1s, W2s, b2s)
    degw, cntw = sc["hist"](dst, batch32, ofeed8, zfeed8)
    h32, h16 = _tc_embed(x, W_embed, b_embed)

    for l in range(NL):
        agga, aggb = sc[f"fused{l}"](h32, h16, gates, src, dst, zfeed32)
        h32, h16 = _tc_node(agga, aggb, degw, h32, h16, Wn[l], bn[l])

    agga, aggb = sc[f"fused{NL}"](h32, h16, gates, src, dst, zfeed32)
    inva, invb = _tc_inv(agga, aggb, degw, Winv, binv)

    poola, poolb = sc["pool"](inva, invb, batch32, zfeed32)
    return _tc_head(poola, poolb, cntw, g1, beta1, W1, b1, g2, beta2, W2, b2)


# speedup vs baseline: 2.2442x; 1.2228x over previous
import functools

import jax
import jax.numpy as jnp
from jax import lax
from jax.experimental import pallas as pl
from jax.experimental.pallas import tpu as pltpu
from jax.experimental.pallas import tpu_sc as plsc

N = 50000
E = 800000
F = 48
INV = 64
NL = 3
G = 4096
EIN = 8
HID = 64

NP = 51200
CHUNK = 200
NSUB = 16
NCORE = 2

def _f32(shape):
    return jax.ShapeDtypeStruct(shape, jnp.float32)


def _sc_gather_pos_body(pos8, src, dst, out_s, out_d, idx_v, rows_v, sem):
    cid = lax.axis_index("c")
    sid = lax.axis_index("s")
    nchunks = E // CHUNK

    def body(j):
        chunk = sid + NSUB * j

        @pl.when(chunk < nchunks)
        def _():
            base = chunk * CHUNK

            @pl.when(cid == 0)
            def _():
                pltpu.sync_copy(src.at[pl.ds(base, CHUNK)], idx_v)
                pltpu.async_copy(pos8.at[idx_v], rows_v, sem).wait()
                pltpu.sync_copy(rows_v, out_s.at[pl.ds(base, CHUNK)])

            @pl.when(cid == 1)
            def _():
                pltpu.sync_copy(dst.at[pl.ds(base, CHUNK)], idx_v)
                pltpu.async_copy(pos8.at[idx_v], rows_v, sem).wait()
                pltpu.sync_copy(rows_v, out_d.at[pl.ds(base, CHUNK)])

    pl.loop(0, nchunks // NSUB)(body)


FCH = 200


def _sc_fused_body_maker(l):
    def body(h32, h16, gates, src, dst, zfeed, outa, outb,
             srcv0, srcv1, dstv0, dstv1, hv, hw, gv, msgv, acc,
             sem_i0, sem_i1, sem_g, sem_e, sem_s):
        cid = lax.axis_index("c")
        sid = lax.axis_index("s")
        srcv = (srcv0, srcv1)
        dstv = (dstv0, dstv1)
        sem_i = (sem_i0, sem_i1)
        nch = E // FCH

        def issue_idx(c, p):
            pltpu.async_copy(src.at[pl.ds(c * FCH, FCH)], srcv[p], sem_i[p])
            pltpu.async_copy(dst.at[pl.ds(c * FCH, FCH)], dstv[p], sem_i[p])

        def wait_idx(p):
            pltpu.make_async_copy(src.at[pl.ds(0, FCH)], srcv[p], sem_i[p]).wait()
            pltpu.make_async_copy(dst.at[pl.ds(0, FCH)], dstv[p], sem_i[p]).wait()

        def issue_fetch(c, p):
            pltpu.async_copy(gates.at[l, pl.ds(c * FCH, FCH)], gv, sem_g)

            @pl.when(cid == 0)
            def _():
                pltpu.async_copy(h32.at[srcv[p]], hv, sem_e)

            @pl.when(cid == 1)
            def _():
                pltpu.async_copy(h16.at[srcv[p]], hw, sem_e)

        def wait_fetch():
            pltpu.make_async_copy(gates.at[l, pl.ds(0, FCH)], gv, sem_g).wait()

            @pl.when(cid == 0)
            def _():
                pltpu.make_async_copy(h32.at[pl.ds(0, FCH)], hv, sem_e).wait()

            @pl.when(cid == 1)
            def _():
                pltpu.make_async_copy(h16.at[pl.ds(0, FCH)], hw, sem_e).wait()

        def wait_scatter():
            pltpu.make_async_copy(zfeed.at[pl.ds(0, FCH)], msgv, sem_s).wait()

        def zbody(j):
            chunk = sid + NSUB * j
            pltpu.sync_copy(zfeed.at[pl.ds(0, FCH)],
                            acc.at[pl.ds(chunk * FCH, FCH)])

        pltpu.sync_copy(zfeed.at[pl.ds(0, FCH)], msgv)
        pl.loop(0, NP // FCH // NSUB)(zbody)
        plsc.subcore_barrier()

        issue_idx(sid, 0)
        wait_idx(0)
        issue_fetch(sid, 0)

        def step(t):
            for s in (0, 1):
                j = 2 * t + s
                c = sid + NSUB * j
                p = s

                @pl.when(j > 0)
                def _():
                    wait_scatter()

                wait_fetch()

                @pl.when(c + NSUB < nch)
                def _():
                    issue_idx(c + NSUB, 1 - p)

                @pl.when(cid == 0)
                def _():
                    def mul0(k):
                        msgv[k, pl.ds(0, 16)] = (hv[k, pl.ds(0, 16)]
                                                 * gv[k, pl.ds(0, 16)])
                        msgv[k, pl.ds(16, 16)] = (hv[k, pl.ds(16, 16)]
                                                  * gv[k, pl.ds(16, 16)])

                    pl.loop(0, FCH, unroll=8)(mul0)

                @pl.when(cid == 1)
                def _():
                    def mul1(k):
                        msgv[k, pl.ds(0, 16)] = (hw[k, pl.ds(0, 16)]
                                                 * gv[k, pl.ds(32, 16)])

                    pl.loop(0, FCH, unroll=8)(mul1)

                pltpu.async_copy(msgv, acc.at[dstv[p]], sem_s, add=True)

                @pl.when(c + NSUB < nch)
                def _():
                    wait_idx(1 - p)
                    issue_fetch(c + NSUB, 1 - p)

        pl.loop(0, nch // NSUB // 2)(step)
        wait_scatter()
        plsc.subcore_barrier()

        rows = NP // NSUB
        rbase = sid * rows

        @pl.when(cid == 0)
        def _():
            pltpu.sync_copy(acc.at[pl.ds(rbase, rows)], outa.at[pl.ds(rbase, rows)])

        @pl.when(cid == 1)
        def _():
            pltpu.sync_copy(acc.at[pl.ds(rbase, rows)], outb.at[pl.ds(rbase, rows)])

    return body


def _sc_hist_body(dst, batch, ofeed, zfeed8, deg_out, cnt_out, idx_v, ones_v, acc, sem):
    cid = lax.axis_index("c")
    sid = lax.axis_index("s")

    def zbody(j):
        chunk = sid + NSUB * j

        @pl.when(cid == 0)
        def _():
            pltpu.sync_copy(zfeed8.at[pl.ds(0, CHUNK)],
                            acc.at[pl.ds(chunk * CHUNK, CHUNK)])

        @pl.when(jnp.logical_and(cid == 1, chunk < G // CHUNK + 1))
        def _():
            @pl.when(chunk * CHUNK < G - CHUNK + 1)
            def _():
                pltpu.sync_copy(zfeed8.at[pl.ds(0, CHUNK)],
                                acc.at[pl.ds(chunk * CHUNK, CHUNK)])

            @pl.when(chunk == G // CHUNK)
            def _():
                pltpu.sync_copy(zfeed8.at[pl.ds(0, G % CHUNK)],
                                acc.at[pl.ds(G - (G % CHUNK), G % CHUNK)])

    pl.loop(0, NP // CHUNK // NSUB)(zbody)
    pltpu.sync_copy(ofeed, ones_v)
    plsc.subcore_barrier()

    ne = E // CHUNK
    nn = N // CHUNK

    def body(j):
        chunk = sid + NSUB * j

        @pl.when(cid == 0)
        def _():
            pltpu.sync_copy(dst.at[pl.ds(chunk * CHUNK, CHUNK)], idx_v)
            pltpu.sync_copy(ones_v, acc.at[idx_v], add=True)

        @pl.when(jnp.logical_and(cid == 1, chunk < nn))
        def _():
            pltpu.sync_copy(batch.at[pl.ds(chunk * CHUNK, CHUNK)], idx_v)
            pltpu.sync_copy(ones_v, acc.at[idx_v], add=True)

    pl.loop(0, ne // NSUB)(body)
    plsc.subcore_barrier()

    @pl.when(cid == 0)
    def _():
        rows = NP // NSUB
        pltpu.sync_copy(acc.at[pl.ds(sid * rows, rows)],
                        deg_out.at[pl.ds(sid * rows, rows)])

    @pl.when(cid == 1)
    def _():
        rows = G // NSUB
        pltpu.sync_copy(acc.at[pl.ds(sid * rows, rows)],
                        cnt_out.at[pl.ds(sid * rows, rows)])


def _sc_pool_body(inva, invb, batch, zfeed, outa, outb, idx_v, row_v, acc, sem):
    cid = lax.axis_index("c")
    sid = lax.axis_index("s")

    rows = G // NSUB
    pltpu.sync_copy(zfeed.at[pl.ds(0, rows)], acc.at[pl.ds(sid * rows, rows)])
    plsc.subcore_barrier()

    nchunks = N // CHUNK

    def body(j):
        chunk = sid + NSUB * j

        @pl.when(chunk < nchunks)
        def _():
            base = chunk * CHUNK
            pltpu.sync_copy(batch.at[pl.ds(base, CHUNK)], idx_v)

            @pl.when(cid == 0)
            def _():
                pltpu.sync_copy(inva.at[pl.ds(base, CHUNK)], row_v)

            @pl.when(cid == 1)
            def _():
                pltpu.sync_copy(invb.at[pl.ds(base, CHUNK)], row_v)

            pltpu.sync_copy(row_v, acc.at[idx_v], add=True)

    pl.loop(0, (nchunks + NSUB - 1) // NSUB)(body)
    plsc.subcore_barrier()

    @pl.when(cid == 0)
    def _():
        pltpu.sync_copy(acc.at[pl.ds(sid * rows, rows)],
                        outa.at[pl.ds(sid * rows, rows)])

    @pl.when(cid == 1)
    def _():
        pltpu.sync_copy(acc.at[pl.ds(sid * rows, rows)],
                        outb.at[pl.ds(sid * rows, rows)])


@functools.cache
def _sc():
    mesh = plsc.VectorSubcoreMesh(core_axis_name="c", subcore_axis_name="s")
    cp = pltpu.CompilerParams(use_tc_tiling_on_sc=False)
    k = {}
    k["gather_pos"] = pl.kernel(
        _sc_gather_pos_body, mesh=mesh,
        out_type=[_f32((E, 8)), _f32((E, 8))],
        scratch_types=[
            pltpu.VMEM((CHUNK,), jnp.int32),
            pltpu.VMEM((CHUNK, 8), jnp.float32),
            pltpu.SemaphoreType.DMA,
        ], compiler_params=cp)
    for l in range(NL + 1):
        k[f"fused{l}"] = pl.kernel(
            _sc_fused_body_maker(l), mesh=mesh,
            out_type=[_f32((NP, 32)), _f32((NP, 32))],
            scratch_types=[
                pltpu.VMEM((FCH,), jnp.int32),
                pltpu.VMEM((FCH,), jnp.int32),
                pltpu.VMEM((FCH,), jnp.int32),
                pltpu.VMEM((FCH,), jnp.int32),
                pltpu.VMEM((FCH, 32), jnp.float32),
                pltpu.VMEM((FCH, 16), jnp.float32),
                pltpu.VMEM((FCH, F), jnp.float32),
                pltpu.VMEM((FCH, 32), jnp.float32),
                pltpu.VMEM_SHARED((NP, 32), jnp.float32),
                pltpu.SemaphoreType.DMA,
                pltpu.SemaphoreType.DMA,
                pltpu.SemaphoreType.DMA,
                pltpu.SemaphoreType.DMA,
                pltpu.SemaphoreType.DMA,
            ], compiler_params=cp)
    k["hist"] = pl.kernel(
        _sc_hist_body, mesh=mesh,
        out_type=[_f32((NP, 8)), _f32((G, 8))],
        scratch_types=[
            pltpu.VMEM((CHUNK,), jnp.int32),
            pltpu.VMEM((CHUNK, 8), jnp.float32),
            pltpu.VMEM_SHARED((NP, 8), jnp.float32),
            pltpu.SemaphoreType.DMA,
        ], compiler_params=cp)
    k["pool"] = pl.kernel(
        _sc_pool_body, mesh=mesh,
        out_type=[_f32((G, 32)), _f32((G, 32))],
        scratch_types=[
            pltpu.VMEM((CHUNK,), jnp.int32),
            pltpu.VMEM((CHUNK, 32), jnp.float32),
            pltpu.VMEM_SHARED((G, 32), jnp.float32),
            pltpu.SemaphoreType.DMA,
        ], compiler_params=cp)
    return k


def _elu(v):
    return jnp.where(v > 0, v, jnp.exp(jnp.minimum(v, 0.0)) - 1.0)


def _tc_embed_body(x_ref, w_ref, b_ref, o32_ref, o16_ref):
    h = _elu(jnp.dot(x_ref[...], w_ref[...],
                     preferred_element_type=jnp.float32) + b_ref[...])
    o32_ref[...] = h[:, 0:32]
    o16_ref[...] = h[:, 32:48]


def _tc_embed(x, W_embed, b_embed):
    blk = 1000
    return pl.pallas_call(
        _tc_embed_body,
        grid=(N // blk,),
        in_specs=[
            pl.BlockSpec((blk, 5), lambda i: (i, 0)),
            pl.BlockSpec((5, F), lambda i: (0, 0)),
            pl.BlockSpec((1, F), lambda i: (0, 0)),
        ],
        out_specs=[
            pl.BlockSpec((blk, 32), lambda i: (i, 0)),
            pl.BlockSpec((blk, 16), lambda i: (i, 0)),
        ],
        out_shape=[_f32((N, 32)), _f32((N, 16))],
    )(x, W_embed, b_embed.reshape(1, F))


def _tc_gates_body(ps_ref, pd_ref, ea_ref, w1_ref, b1_ref, w2_ref, b2_ref, o_ref):
    delta = pd_ref[:, 0:3] - ps_ref[:, 0:3]
    dist = jnp.sqrt(jnp.sum(delta * delta, axis=-1, keepdims=True) + 1e-12)
    ef = jnp.concatenate([delta, dist, ea_ref[...]], axis=-1)
    for l in range(NL + 1):
        hdn = _elu(jnp.dot(ef, w1_ref[l], preferred_element_type=jnp.float32)
                   + b1_ref[l])
        ew = jnp.dot(hdn, w2_ref[l], preferred_element_type=jnp.float32) + b2_ref[l]
        o_ref[l] = jax.nn.sigmoid(ew)


def _tc_gates(possrc, posdst, edge_attr, W1s, b1s, W2s, b2s):
    blk = 1000
    return pl.pallas_call(
        _tc_gates_body,
        grid=(E // blk,),
        in_specs=[
            pl.BlockSpec((blk, 8), lambda i: (i, 0)),
            pl.BlockSpec((blk, 8), lambda i: (i, 0)),
            pl.BlockSpec((blk, 4), lambda i: (i, 0)),
            pl.BlockSpec((NL + 1, EIN, HID), lambda i: (0, 0, 0)),
            pl.BlockSpec((NL + 1, 1, HID), lambda i: (0, 0, 0)),
            pl.BlockSpec((NL + 1, HID, F), lambda i: (0, 0, 0)),
            pl.BlockSpec((NL + 1, 1, F), lambda i: (0, 0, 0)),
        ],
        out_specs=pl.BlockSpec((NL + 1, blk, F), lambda i: (0, i, 0)),
        out_shape=_f32((NL + 1, E, F)),
    )(possrc, posdst, edge_attr, W1s, b1s.reshape(NL + 1, 1, HID),
      W2s, b2s.reshape(NL + 1, 1, F))


def _tc_node_body(aa_ref, ab_ref, deg_ref, h32_ref, h16_ref, w_ref, b_ref,
                  o32_ref, o16_ref):
    agg = jnp.concatenate([aa_ref[...], ab_ref[:, 0:16]], axis=-1)
    deg = jnp.maximum(deg_ref[:, 0:1], 1.0)
    h = jnp.concatenate([h32_ref[...], h16_ref[...]], axis=-1)
    z = jnp.dot(agg / deg, w_ref[...], preferred_element_type=jnp.float32) + b_ref[...]
    hn = h + _elu(z)
    o32_ref[...] = hn[:, 0:32]
    o16_ref[...] = hn[:, 32:48]


def _tc_node(agga, aggb, degw, h32, h16, Wn_l, bn_l):
    blk = 1000
    return pl.pallas_call(
        _tc_node_body,
        grid=(N // blk,),
        in_specs=[
            pl.BlockSpec((blk, 32), lambda i: (i, 0)),
            pl.BlockSpec((blk, 32), lambda i: (i, 0)),
            pl.BlockSpec((blk, 8), lambda i: (i, 0)),
            pl.BlockSpec((blk, 32), lambda i: (i, 0)),
            pl.BlockSpec((blk, 16), lambda i: (i, 0)),
            pl.BlockSpec((F, F), lambda i: (0, 0)),
            pl.BlockSpec((1, F), lambda i: (0, 0)),
        ],
        out_specs=[
            pl.BlockSpec((blk, 32), lambda i: (i, 0)),
            pl.BlockSpec((blk, 16), lambda i: (i, 0)),
        ],
        out_shape=[_f32((N, 32)), _f32((N, 16))],
    )(agga, aggb, degw, h32, h16, Wn_l, bn_l.reshape(1, F))


def _tc_inv_body(aa_ref, ab_ref, deg_ref, w_ref, b_ref, oa_ref, ob_ref):
    agg = jnp.concatenate([aa_ref[...], ab_ref[:, 0:16]], axis=-1)
    deg = jnp.maximum(deg_ref[:, 0:1], 1.0)
    z = jnp.dot(agg / deg, w_ref[...], preferred_element_type=jnp.float32) + b_ref[...]
    oa_ref[...] = z[:, 0:32]
    ob_ref[...] = z[:, 32:64]


def _tc_inv(agga, aggb, degw, Winv, binv):
    blk = 1000
    return pl.pallas_call(
        _tc_inv_body,
        grid=(N // blk,),
        in_specs=[
            pl.BlockSpec((blk, 32), lambda i: (i, 0)),
            pl.BlockSpec((blk, 32), lambda i: (i, 0)),
            pl.BlockSpec((blk, 8), lambda i: (i, 0)),
            pl.BlockSpec((F, INV), lambda i: (0, 0)),
            pl.BlockSpec((1, INV), lambda i: (0, 0)),
        ],
        out_specs=[
            pl.BlockSpec((blk, 32), lambda i: (i, 0)),
            pl.BlockSpec((blk, 32), lambda i: (i, 0)),
        ],
        out_shape=[_f32((N, 32)), _f32((N, 32))],
    )(agga, aggb, degw, Winv, binv.reshape(1, INV))


def _tc_head_body(pa_ref, pb_ref, cnt_ref, g1_ref, be1_ref, w1_ref, b1_ref,
                  g2_ref, be2_ref, w2_ref, b2_ref, o_ref):
    cnt = jnp.maximum(cnt_ref[:, 0:1], 1.0)
    pooled = jnp.concatenate([pa_ref[...], pb_ref[...]], axis=-1) / cnt

    def bnorm(z, g, b):
        m = jnp.mean(z, axis=0, keepdims=True)
        v = jnp.mean((z - m) ** 2, axis=0, keepdims=True)
        return g * (z - m) / jnp.sqrt(v + 1e-5) + b

    z = _elu(bnorm(pooled, g1_ref[...], be1_ref[...]))
    z = jnp.dot(z, w1_ref[...], preferred_element_type=jnp.float32) + b1_ref[...]
    z = _elu(bnorm(z, g2_ref[...], be2_ref[...]))
    o_ref[...] = jnp.dot(z, w2_ref[...], preferred_element_type=jnp.float32) + b2_ref[...]


def _tc_head(poola, poolb, cntw, g1, beta1, W1, b1, g2, beta2, W2, b2):
    full = lambda s: pl.BlockSpec(s, lambda: tuple(0 for _ in s))
    return pl.pallas_call(
        _tc_head_body,
        in_specs=[
            full((G, 32)), full((G, 32)), full((G, 8)),
            full((1, INV)), full((1, INV)), full((INV, INV)), full((1, INV)),
            full((1, INV)), full((1, INV)), full((INV, 1)), full((1, 1)),
        ],
        out_specs=full((G, 1)),
        out_shape=_f32((G, 1)),
    )(poola, poolb, cntw, g1.reshape(1, INV), beta1.reshape(1, INV), W1,
      b1.reshape(1, INV), g2.reshape(1, INV), beta2.reshape(1, INV), W2,
      b2.reshape(1, 1))


def kernel(x, pos, edge_index, edge_attr, batch, W_embed, b_embed, We1, be1,
           We2, be2, Wn, bn, Wie1, bie1, Wie2, bie2, Winv, binv, g1, beta1,
           W1, b1, g2, beta2, W2, b2):
    src = edge_index[0].astype(jnp.int32)
    dst = edge_index[1].astype(jnp.int32)
    batch32 = batch.astype(jnp.int32)

    pos8 = jnp.concatenate([pos, jnp.zeros((N, 5), jnp.float32)], axis=1)
    W1s = jnp.concatenate([We1, Wie1[None]], axis=0)
    b1s = jnp.concatenate([be1, bie1[None]], axis=0)
    W2s = jnp.concatenate([We2, Wie2[None]], axis=0)
    b2s = jnp.concatenate([be2, bie2[None]], axis=0)

    zfeed32 = jnp.zeros((400, 32), jnp.float32)
    zfeed8 = jnp.zeros((400, 8), jnp.float32)
    ofeed8 = jnp.ones((CHUNK, 8), jnp.float32)

    sc = _sc()
    possrc, posdst = sc["gather_pos"](pos8, src, dst)
    gates = _tc_gates(possrc, posdst, edge_attr, W1s, b1s, W2s, b2s)
    degw, cntw = sc["hist"](dst, batch32, ofeed8, zfeed8)
    h32, h16 = _tc_embed(x, W_embed, b_embed)

    for l in range(NL):
        agga, aggb = sc[f"fused{l}"](h32, h16, gates, src, dst, zfeed32)
        h32, h16 = _tc_node(agga, aggb, degw, h32, h16, Wn[l], bn[l])

    agga, aggb = sc[f"fused{NL}"](h32, h16, gates, src, dst, zfeed32)
    inva, invb = _tc_inv(agga, aggb, degw, Winv, binv)

    poola, poolb = sc["pool"](inva, invb, batch32, zfeed32)
    return _tc_head(poola, poolb, cntw, g1, beta1, W1, b1, g2, beta2, W2, b2)
